# baseline (speedup 1.0000x reference)
import functools

import jax
import jax.numpy as jnp
from jax.experimental import pallas as pl
from jax.experimental.pallas import tpu as pltpu


_K_CAP = 6144
_TA = 2304
_TK = 512


def _dense_kernel(clsf_ref, regf_ref, annr4f_ref, st16_ref, st32_ref,
                  out_ref, *, c):
    i = pl.program_id(0)

    @pl.when(i == 0)
    def _():
        out_ref[0] = 0.0
        out_ref[1] = 0.0
        out_ref[1] = 0.0
        out_ref[2] = 0.0
        out_ref[3] = 0.0

    ng = 128 // c
    lane = jax.lax.broadcasted_iota(jnp.int32, (128, ng), 0)
    grp = jax.lax.broadcasted_iota(jnp.int32, (128, ng), 1)
    e16 = jnp.where(lane // c == grp, 1.0, 0.0)
    lane2 = jax.lax.broadcasted_iota(jnp.int32, (128, 32), 0)
    grp2 = jax.lax.broadcasted_iota(jnp.int32, (128, 32), 1)
    e32 = jnp.where(lane2 // 4 == grp2, 1.0, 0.0)

    p = jnp.clip(clsf_ref[...], 1e-4, 1.0 - 1e-4)
    f0 = 0.75 * p * p * (-jnp.log(1.0 - p))
    gs_f = jax.lax.dot_general(f0, e16, (((1,), (0,)), ((), ())),
                               preferred_element_type=jnp.float32)
    st16 = st16_ref[...]
    vm16 = st16 != -1.0
    out_ref[0] += jnp.sum(jnp.where(vm16, gs_f, 0.0))

    d = jnp.abs(regf_ref[...] - annr4f_ref[...])
    l = jnp.where(d < 3.0, 0.5 * d * d / 3.0, d - 1.5)
    gs_l = jax.lax.dot_general(l, e32, (((1,), (0,)), ((), ())),
                               preferred_element_type=jnp.float32)
    vm32 = st32_ref[...] != -1.0
    out_ref[1] += jnp.sum(jnp.where(vm32, gs_l, 0.0))

    out_ref[2] += jnp.sum(jnp.where(st16 == 1.0, 1.0, 0.0))
    out_ref[3] += jnp.sum(jnp.where(vm16, 1.0, 0.0))


def _rodrigues_rows(rx, ry, rz):
    theta = jnp.sqrt(rx * rx + ry * ry + rz * rz)
    safe = jnp.maximum(theta, 1e-8)
    ax = rx / safe
    ay = ry / safe
    az = rz / safe
    ct = jnp.cos(theta)
    st = jnp.sin(theta)
    oc = 1.0 - ct
    r00 = 1.0 - oc * (ay * ay + az * az)
    r01 = st * (-az) + oc * (ax * ay)
    r02 = st * ay + oc * (ax * az)
    r10 = st * az + oc * (ax * ay)
    r11 = 1.0 - oc * (ax * ax + az * az)
    r12 = st * (-ax) + oc * (ay * az)
    r20 = st * (-ay) + oc * (ax * az)
    r21 = st * ax + oc * (ay * az)
    r22 = 1.0 - oc * (ax * ax + ay * ay)
    return (r00, r01, r02, r10, r11, r12, r20, r21, r22)


def _trans_kernel(packed_ref, mp_ref, out_ref, scr_ref, *, npts):
    i = pl.program_id(0)

    @pl.when(i == 0)
    def _():
        out_ref[0] = 0.0
        out_ref[1] = 0.0

    ci = packed_ref[13:14, :].astype(jnp.int32)
    onehot = jnp.where(
        jax.lax.broadcasted_iota(jnp.int32, (8, ci.shape[1]), 0) == ci,
        1.0, 0.0)

    gx = jax.lax.dot_general(mp_ref[0:npts, :], onehot,
                             (((1,), (0,)), ((), ())),
                             preferred_element_type=jnp.float32)
    gy = jax.lax.dot_general(mp_ref[npts:2 * npts, :], onehot,
                             (((1,), (0,)), ((), ())),
                             preferred_element_type=jnp.float32)
    gz = jax.lax.dot_general(mp_ref[2 * npts:3 * npts, :], onehot,
                             (((1,), (0,)), ((), ())),
                             preferred_element_type=jnp.float32)

    p00, p01, p02, p10, p11, p12, p20, p21, p22 = _rodrigues_rows(
        packed_ref[0:1, :], packed_ref[1:2, :], packed_ref[2:3, :])
    t00, t01, t02, t10, t11, t12, t20, t21, t22 = _rodrigues_rows(
        packed_ref[6:7, :], packed_ref[7:8, :], packed_ref[8:9, :])

    tpx = p00 * gx + p01 * gy + p02 * gz + packed_ref[3:4, :]
    tpy = p10 * gx + p11 * gy + p12 * gz + packed_ref[4:5, :]
    tpz = p20 * gx + p21 * gy + p22 * gz + packed_ref[5:6, :]
    ttx = t00 * gx + t01 * gy + t02 * gz + packed_ref[9:10, :]
    tty = t10 * gx + t11 * gy + t12 * gz + packed_ref[10:11, :]
    ttz = t20 * gx + t21 * gy + t22 * gz + packed_ref[11:12, :]

    dx = tpx - ttx
    dy = tpy - tty
    dz = tpz - ttz
    d_asym = jnp.mean(jnp.sqrt(dx * dx + dy * dy + dz * dz),
                      axis=0, keepdims=True)

    sq_tp = tpx * tpx + tpy * tpy + tpz * tpz
    sq_tt = ttx * ttx + tty * tty + ttz * ttz

    scr_ref[0 * npts:1 * npts, :] = ttx
    scr_ref[1 * npts:2 * npts, :] = tty
    scr_ref[2 * npts:3 * npts, :] = ttz
    scr_ref[3 * npts:4 * npts, :] = sq_tt

    def body(m, mins):
        txm = scr_ref[pl.ds(m, 1), :]
        tym = scr_ref[pl.ds(npts + m, 1), :]
        tzm = scr_ref[pl.ds(2 * npts + m, 1), :]
        sqm = scr_ref[pl.ds(3 * npts + m, 1), :]
        d2 = sq_tp + sqm - 2.0 * (tpx * txm + tpy * tym + tpz * tzm)
        return jnp.minimum(mins, d2)

    mins = jax.lax.fori_loop(
        0, npts, body,
        jnp.full(sq_tp.shape, jnp.inf, dtype=jnp.float32))
    d_sym = jnp.mean(jnp.sqrt(jnp.maximum(mins, 1e-12)),
                     axis=0, keepdims=True)

    per = jnp.where(packed_ref[12:13, :] > 0.5, d_sym, d_asym)
    vk = packed_ref[14:15, :]
    per = per * vk
    out_ref[0] += jnp.sum(per)

    pc = jnp.clip(packed_ref[15:16, :], 1e-4, 1.0 - 1e-4)
    f1 = 0.25 * (1.0 - pc) * (1.0 - pc) * (-jnp.log(pc))
    f0c = 0.75 * pc * pc * (-jnp.log(1.0 - pc))
    out_ref[1] += jnp.sum(vk * (f1 - f0c))


def kernel(classification, regression, rotation, translation,
           annotations_cls, annotations_reg, annotations_trans,
           model_points):
    b, a, c = classification.shape
    ba = b * a
    npts = model_points.shape[1]

    ta = ba
    for t in range(min(ba, 32768), 31, -1):
        if ba % t == 0 and t % 256 == 0:
            ta = t
            break
    rc = ta * c // 128
    rr = ta * 4 // 128
    r16 = ta // (128 // c)
    clsf = classification.reshape(ba * c // 128, 128)
    regf = regression.reshape(ba * 4 // 128, 128)
    annr4f = annotations_reg[:, :, 0:4].reshape(ba * 4 // 128, 128)
    st = annotations_cls[:, :, c]
    st16 = st.reshape(ba // (128 // c), 128 // c)
    st32 = st.reshape(ba // 32, 32)

    sums = pl.pallas_call(
        functools.partial(_dense_kernel, c=c),
        grid=(ba // ta,),
        in_specs=[
            pl.BlockSpec((rc, 128), lambda i: (i, 0)),
            pl.BlockSpec((rr, 128), lambda i: (i, 0)),
            pl.BlockSpec((rr, 128), lambda i: (i, 0)),
            pl.BlockSpec((r16, 128 // c), lambda i: (i, 0)),
            pl.BlockSpec((ta // 32, 32), lambda i: (i, 0)),
        ],
        out_specs=pl.BlockSpec(memory_space=pltpu.SMEM),
        out_shape=jax.ShapeDtypeStruct((4,), jnp.float32),
        compiler_params=pltpu.CompilerParams(
            dimension_semantics=("arbitrary",)),
    )(clsf, regf, annr4f, st16, st32)

    f_sum, l_sum, num_pos, num_valid = sums[0], sums[1], sums[2], sums[3]

    k_cap = min(_K_CAP, ((ba + _TK - 1) // _TK) * _TK)
    posf = (annotations_trans[:, :, 8] == 1.0).reshape(ba)
    (idx,) = jnp.nonzero(posf, size=k_cap, fill_value=ba - 1)
    cnt = jnp.sum(posf.astype(jnp.int32))
    valid_k = (jnp.arange(k_cap) < cnt).astype(jnp.float32)

    bi = idx // a
    ai = idx % a
    rp = rotation[bi, ai]
    tp = translation[bi, ai]
    at = annotations_trans[bi, ai]
    pc = classification[bi, ai, at[:, 7].astype(jnp.int32)]
    packed = jnp.concatenate(
        [rp, tp, at[:, 0:8], valid_k[:, None], pc[:, None]],
        axis=1).T

    mp_t = jnp.transpose(model_points, (2, 1, 0)).reshape(3 * npts,
                                                          model_points.shape[0])

    grid_b = k_cap // _TK
    t_sum = pl.pallas_call(
        functools.partial(_trans_kernel, npts=npts),
        grid=(grid_b,),
        in_specs=[
            pl.BlockSpec((16, _TK), lambda i: (0, i)),
            pl.BlockSpec((3 * npts, model_points.shape[0]), lambda i: (0, 0)),
        ],
        out_specs=pl.BlockSpec(memory_space=pltpu.SMEM),
        out_shape=jax.ShapeDtypeStruct((2,), jnp.float32),
        scratch_shapes=[pltpu.VMEM((4 * npts, _TK), jnp.float32)],
        compiler_params=pltpu.CompilerParams(
            dimension_semantics=("arbitrary",)),
    )(packed, mp_t)

    denom_pos = jnp.maximum(num_pos, 1.0)
    cls_loss = (f_sum + t_sum[1]) / denom_pos
    reg_loss = l_sum / (num_valid * 4.0)
    trans_loss = t_sum[0] / denom_pos
    total = reg_loss + cls_loss + 0.02 * trans_loss
    return (total, cls_loss, reg_loss, trans_loss)
